# Initial kernel scaffold; baseline (speedup 1.0000x reference)
import jax
import jax.numpy as jnp
from jax.experimental import pallas as pl


def kernel(feature, xyz, Wq, bq, Wk, bk, Wv, bv, Wc, bc):
    raise NotImplementedError("write your pallas kernel here")


# speedup vs baseline: 14.5312x; 14.5312x over previous
import functools

import jax
import jax.numpy as jnp
from jax import lax
from jax.experimental import pallas as pl

BLK = 128
K_NN = 16
K_Q = 8
TD = 64


def _knn_proj_kernel(xyz_blk_ref, xyz_all_ref, x_blk_ref, w6_ref, b6_ref,
                     idx_ref, p6_ref, *, n_total):
    nb = pl.program_id(1)
    base = nb * BLK

    xq = xyz_blk_ref[0]
    xa = xyz_all_ref[0]
    d2 = jnp.zeros((BLK, n_total), dtype=jnp.float32)
    for d in range(3):
        diff = xq[d][:, None] - xa[d][None, :]
        d2 = d2 + diff * diff

    iota_n = lax.broadcasted_iota(jnp.int32, (BLK, n_total), 1)
    iota_m = lax.broadcasted_iota(jnp.int32, (BLK, n_total), 0)
    big = jnp.float32(3.4e38)
    d2 = jnp.where(iota_n == iota_m + base, big, d2)

    rows = []
    for _ in range(K_NN):
        m = jnp.min(d2, axis=1)
        cand = jnp.where(d2 <= m[:, None], iota_n, jnp.int32(n_total))
        a = jnp.min(cand, axis=1)
        rows.append(a)
        d2 = jnp.where(iota_n == a[:, None], big, d2)
    idx_ref[0] = jnp.stack(rows, axis=0)

    x = x_blk_ref[0]
    w6 = w6_ref[...]
    p6 = jnp.dot(w6, x, preferred_element_type=jnp.float32) + b6_ref[...]
    p6_ref[0] = p6


def _attn_kernel(p6_ref, idx_ref, feat_ref, wc_ref, bc_ref, out_ref,
                 *, n_total):
    nb = pl.program_id(1)
    base = nb * BLK

    p_nbr = p6_ref[0, 0:192, :]
    qc = p6_ref[0, 192:256, pl.ds(base, BLK)]
    kc = p6_ref[0, 256:320, pl.ds(base, BLK)]
    vc = p6_ref[0, 320:384, pl.ds(base, BLK)]
    idxb = idx_ref[0]

    iota_p = lax.broadcasted_iota(jnp.int32, (n_total, BLK), 0)

    gq, gk, gv = [], [], []
    for j in range(K_NN):
        onehot = (iota_p == idxb[j][None, :]).astype(jnp.float32)
        g = jnp.dot(p_nbr, onehot, preferred_element_type=jnp.float32)
        if j < K_Q:
            gq.append(g[0:TD] + qc)
        gk.append(g[TD:2 * TD] + kc)
        gv.append(g[2 * TD:3 * TD] + vc)

    arows = []
    for i in range(K_Q):
        arows.append(jnp.stack(
            [jnp.sum(gq[i] * gk[j], axis=0) for j in range(K_NN)], axis=0))
    attn = jnp.stack(arows, axis=0)
    mx = jnp.max(attn, axis=1, keepdims=True)
    e = jnp.exp(attn - mx)
    s = jnp.sum(e, axis=1, keepdims=True)
    w = jnp.sum(e / s, axis=0)

    res = jnp.zeros((TD, BLK), dtype=jnp.float32)
    for j in range(K_NN):
        res = res + gv[j] * w[j][None, :]

    out = jnp.dot(wc_ref[...], res, preferred_element_type=jnp.float32)
    out_ref[0] = out + bc_ref[...] + feat_ref[0]


def kernel(feature, xyz, Wq, bq, Wk, bk, Wv, bv, Wc, bc):
    B, C, N = feature.shape
    nblk = N // BLK

    def split(W):
        return W[:, 0:3], W[:, 3:6], W[:, 6:6 + C], W[:, 6 + C:6 + 2 * C]

    qa, qb, qc_, qd = split(Wq)
    ka, kb, kc_, kd = split(Wk)
    va, vb, vc_, vd = split(Wv)
    w_nbr = jnp.concatenate([
        jnp.concatenate([qa, qc_], axis=1),
        jnp.concatenate([ka, kc_], axis=1),
        jnp.concatenate([va, vc_], axis=1),
    ], axis=0)
    w_ctr = jnp.concatenate([
        jnp.concatenate([qb - qa, qd - qc_], axis=1),
        jnp.concatenate([kb - ka, kd - kc_], axis=1),
        jnp.concatenate([vb - va, vd - vc_], axis=1),
    ], axis=0)
    w6 = jnp.concatenate([w_nbr, w_ctr], axis=0)
    b6 = jnp.concatenate(
        [jnp.zeros((192,), jnp.float32), bq, bk, bv])[:, None]
    x_in = jnp.concatenate([xyz, feature], axis=1)

    idx, p6 = pl.pallas_call(
        functools.partial(_knn_proj_kernel, n_total=N),
        grid=(B, nblk),
        in_specs=[
            pl.BlockSpec((1, 3, BLK), lambda b, n: (b, 0, n)),
            pl.BlockSpec((1, 3, N), lambda b, n: (b, 0, 0)),
            pl.BlockSpec((1, 3 + C, BLK), lambda b, n: (b, 0, n)),
            pl.BlockSpec((384, 3 + C), lambda b, n: (0, 0)),
            pl.BlockSpec((384, 1), lambda b, n: (0, 0)),
        ],
        out_specs=[
            pl.BlockSpec((1, K_NN, BLK), lambda b, n: (b, 0, n)),
            pl.BlockSpec((1, 384, BLK), lambda b, n: (b, 0, n)),
        ],
        out_shape=[
            jax.ShapeDtypeStruct((B, K_NN, N), jnp.int32),
            jax.ShapeDtypeStruct((B, 384, N), jnp.float32),
        ],
    )(xyz, xyz, x_in, w6, b6)

    out = pl.pallas_call(
        functools.partial(_attn_kernel, n_total=N),
        grid=(B, nblk),
        in_specs=[
            pl.BlockSpec((1, 384, N), lambda b, n: (b, 0, 0)),
            pl.BlockSpec((1, K_NN, BLK), lambda b, n: (b, 0, n)),
            pl.BlockSpec((1, C, BLK), lambda b, n: (b, 0, n)),
            pl.BlockSpec((C, TD), lambda b, n: (0, 0)),
            pl.BlockSpec((C, 1), lambda b, n: (0, 0)),
        ],
        out_specs=pl.BlockSpec((1, C, BLK), lambda b, n: (b, 0, n)),
        out_shape=jax.ShapeDtypeStruct((B, C, N), jnp.float32),
    )(p6, idx, feature, Wc, bc[:, None])

    return out
